# baseline (speedup 1.0000x reference)
import functools
import math

import jax
import jax.numpy as jnp
import numpy as np
from jax import lax
from jax.experimental import pallas as pl
from jax.experimental.pallas import tpu as pltpu
from jax.experimental.pallas import tpu_sc as plsc

N_ATOMS = 10000
N_PAIRS = 320000
F = 128
R = 64
NB = 5
NRI = 3
NRF = 2
_LOG2 = math.log(2.0)

_NC = 2
_NS = 16
_NW = _NC * _NS
_C = 80
_EW = N_PAIRS // _NW
_NCH = _EW // _C
_NQ = (_NCH - 1) // 4
_SU = 624
_TAIL = N_ATOMS - _SU * _NS

_PERM = np.empty((F,), dtype=np.int32)
for _l in range(F // 32):
    for _k in range(16):
        _PERM[32 * _l + 2 * _k] = 32 * _l + _k
        _PERM[32 * _l + 2 * _k + 1] = 32 * _l + 16 + _k


def _ssp(v):
    return jnp.maximum(v, 0.0) + jnp.log1p(jnp.exp(-jnp.abs(v))) - _LOG2


_EG_CHUNK = 6400


def _edge_g_body(rbf_ref, cut_ref, *refs):
    w_refs = refs[:NB]
    g_refs = refs[NB:]
    d = rbf_ref[...] * cut_ref[...]
    for b in range(NB):
        g_refs[b][...] = jnp.dot(d, w_refs[b][...],
                                 preferred_element_type=jnp.float32)


def _edge_g_call(rbfs, cutoffs, w_list):
    grid = (N_PAIRS // _EG_CHUNK,)
    return pl.pallas_call(
        _edge_g_body,
        grid=grid,
        in_specs=[pl.BlockSpec((_EG_CHUNK, R), lambda i: (i, 0)),
                  pl.BlockSpec((_EG_CHUNK, 1), lambda i: (i, 0))] +
                 [pl.BlockSpec((R, F), lambda i: (0, 0))] * NB,
        out_specs=[pl.BlockSpec((_EG_CHUNK, F), lambda i: (i, 0))] * NB,
        out_shape=[jax.ShapeDtypeStruct((N_PAIRS, F), jnp.float32)] * NB,
    )(rbfs, cutoffs.reshape(N_PAIRS, 1), *w_list)


def _atom_pre_body(x_ref, wi_ref, bi_ref, wj_ref, bj_ref, xi_ref, y_ref):
    xa = _ssp(x_ref[...])
    xi_ref[...] = _ssp(jnp.dot(xa, wi_ref[...],
                               preferred_element_type=jnp.float32) + bi_ref[...])
    y_ref[...] = _ssp(jnp.dot(xa, wj_ref[...],
                              preferred_element_type=jnp.float32) + bj_ref[...])


def _atom_pre_call(x, p):
    return pl.pallas_call(
        _atom_pre_body,
        out_shape=(jax.ShapeDtypeStruct((N_ATOMS, F), jnp.float32),
                   jax.ShapeDtypeStruct((N_ATOMS, F), jnp.float32)),
    )(x, p["dense_i"]["W"], p["dense_i"]["b"].reshape(1, F),
      p["dense_j"]["W"], p["dense_j"]["b"].reshape(1, F))


def _atom_post_body(m01_ref, xi_ref, x_ref, *refs):
    out_ref = refs[-1]
    w = [r[...] for r in refs[:-1]]
    k = 0
    m = m01_ref[0] + m01_ref[1] + xi_ref[...]
    for _ in range(NRI):
        w1, b1, w2, b2 = w[k], w[k + 1], w[k + 2], w[k + 3]
        k += 4
        ym = _ssp(m)
        ym = _ssp(jnp.dot(ym, w1, preferred_element_type=jnp.float32) + b1)
        m = m + jnp.dot(ym, w2, preferred_element_type=jnp.float32) + b2
    wo, bo, u = w[k], w[k + 1], w[k + 2]
    k += 3
    m = _ssp(m)
    x = u * x_ref[...] + jnp.dot(m, wo, preferred_element_type=jnp.float32) + bo
    for _ in range(NRF):
        w1, b1, w2, b2 = w[k], w[k + 1], w[k + 2], w[k + 3]
        k += 4
        yx = _ssp(x)
        yx = _ssp(jnp.dot(yx, w1, preferred_element_type=jnp.float32) + b1)
        x = x + jnp.dot(yx, w2, preferred_element_type=jnp.float32) + b2
    out_ref[...] = x


def _atom_post_call(m01, xi, x, p):
    ws = []
    for rp in p["res_int"]:
        ws += [rp["d1"]["W"], rp["d1"]["b"].reshape(1, F),
               rp["d2"]["W"], rp["d2"]["b"].reshape(1, F)]
    ws += [p["dense_out"]["W"], p["dense_out"]["b"].reshape(1, F),
           p["u"].reshape(1, F)]
    for rp in p["res_feat"]:
        ws += [rp["d1"]["W"], rp["d1"]["b"].reshape(1, F),
               rp["d2"]["W"], rp["d2"]["b"].reshape(1, F)]
    return pl.pallas_call(
        _atom_post_body,
        out_shape=jax.ShapeDtypeStruct((N_ATOMS, F), jnp.float32),
    )(m01, xi, x, *ws)


@functools.cache
def _edge_pass_kernel():
    mesh = plsc.VectorSubcoreMesh(core_axis_name="c", subcore_axis_name="s")

    @functools.partial(
        pl.kernel,
        mesh=mesh,
        out_type=jax.ShapeDtypeStruct((_NC, N_ATOMS, F), jnp.float32),
        scratch_types=[
            [pltpu.VMEM((_C,), jnp.int32)] * 4,
            [pltpu.VMEM((_C,), jnp.int32)] * 4,
            [pltpu.VMEM((_C, F), jnp.float32)] * 2,
            [pltpu.VMEM((_C, F), jnp.float32)] * 2,
            pltpu.VMEM_SHARED((N_ATOMS, F), jnp.float32),
            [pltpu.SemaphoreType.DMA] * 4,
            [pltpu.SemaphoreType.DMA] * 2,
            [pltpu.SemaphoreType.DMA] * 2,
            [pltpu.SemaphoreType.DMA] * 2,
        ],
    )
    def _edge_pass(g_hbm, y_hbm, ii_hbm, ij_hbm, out_hbm,
                   ii_c, ij_c, g_v, rows_v, m_sh,
                   sem_i, sem_g, sem_r, sem_s):
        return _edge_pass_body(g_hbm, y_hbm, ii_hbm, ij_hbm, out_hbm,
                               ii_c, ij_c, g_v, rows_v, m_sh,
                               sem_i, sem_g, sem_r, sem_s)

    return _edge_pass


def _mul_row(rows, g, i):
    for l in range(F // 16):
        s0 = pl.ds(l * 16, 16)
        rows[i, s0] = rows[i, s0] * g[i, s0]


def _edge_pass_body(g_hbm, y_hbm, ii_hbm, ij_hbm, out_hbm,
                    ii_c, ij_c, g_v, rows_v, m_sh,
                    sem_i, sem_g, sem_r, sem_s):
    cid = lax.axis_index("c")
    sid = lax.axis_index("s")
    wid = cid * _NS + sid
    ebase = wid * _EW

    def _issue_idx(u, s):
        pltpu.async_copy(ii_hbm.at[pl.ds(ebase + u * _C, _C)], ii_c[s],
                         sem_i[s])
        pltpu.async_copy(ij_hbm.at[pl.ds(ebase + u * _C, _C)], ij_c[s],
                         sem_i[s])

    def _wait_idx(s):
        pltpu.make_async_copy(ii_hbm.at[pl.ds(0, _C)], ii_c[s],
                              sem_i[s]).wait()
        pltpu.make_async_copy(ij_hbm.at[pl.ds(0, _C)], ij_c[s],
                              sem_i[s]).wait()

    def _issue_data(u, b, s):
        pltpu.async_copy(g_hbm.at[pl.ds(ebase + u * _C, _C)], g_v[b],
                         sem_g[b])
        pltpu.async_copy(y_hbm.at[ij_c[s]], rows_v[b], sem_r[b])

    def _wait_data(b, s):
        pltpu.make_async_copy(g_hbm.at[pl.ds(0, _C)], g_v[b],
                              sem_g[b]).wait()
        pltpu.make_async_copy(y_hbm.at[ij_c[s]], rows_v[b],
                              sem_r[b]).wait()

    for s in range(3):
        _issue_idx(s, s)

    zero = jnp.zeros((16,), jnp.float32)

    def _zrow(i, _):
        for b in range(2):
            for l in range(F // 16):
                rows_v[b][i, pl.ds(l * 16, 16)] = zero
        return 0

    lax.fori_loop(0, _C, _zrow, 0)
    for r in range(7):
        pltpu.sync_copy(rows_v[r % 2],
                        m_sh.at[pl.ds(sid * _SU + r * _C, _C)])
    pltpu.sync_copy(rows_v[1].at[pl.ds(0, _SU - 7 * _C)],
                    m_sh.at[pl.ds(sid * _SU + 7 * _C, _SU - 7 * _C)])

    @pl.when(sid == _NS - 1)
    def _ztail():
        pltpu.sync_copy(rows_v[0].at[pl.ds(0, _TAIL)],
                        m_sh.at[pl.ds(_SU * _NS, _TAIL)])

    _wait_idx(0)
    _issue_data(0, 0, 0)

    plsc.subcore_barrier()

    def _step(k, b4):
        u = 4 * k + b4
        b = b4 % 2
        bo = 1 - b
        _wait_data(b, b4)

        def _retire():
            pltpu.make_async_copy(rows_v[bo], m_sh.at[ii_c[(b4 + 3) % 4]],
                                  sem_s[bo]).wait()

        if b4 == 0:
            pl.when(k > 0)(_retire)
        else:
            _retire()

        _wait_idx((b4 + 1) % 4)
        _issue_data(u + 1, bo, (b4 + 1) % 4)

        def _prefetch_idx():
            _issue_idx(u + 3, (b4 + 3) % 4)

        if b4 < 2:
            _prefetch_idx()
        else:
            pl.when(k < _NQ - 1)(_prefetch_idx)

        @plsc.parallel_loop(0, _C, 1, unroll=2)
        def _mul(i):
            _mul_row(rows_v[b], g_v[b], i)

        pltpu.async_copy(rows_v[b], m_sh.at[ii_c[b4]], sem_s[b], add=True)

    def _quad(k, _):
        for b4 in range(4):
            _step(k, b4)
        return 0

    lax.fori_loop(0, _NQ, _quad, 0)

    pltpu.make_async_copy(rows_v[1], m_sh.at[ii_c[3]], sem_s[1]).wait()
    _wait_data(0, 0)

    def _mul_last(i, _):
        _mul_row(rows_v[0], g_v[0], i)
        return 0

    lax.fori_loop(0, _C, _mul_last, 0)
    pltpu.sync_copy(rows_v[0], m_sh.at[ii_c[0]], add=True)

    plsc.subcore_barrier()

    pltpu.sync_copy(m_sh.at[pl.ds(sid * _SU, _SU)],
                    out_hbm.at[cid, pl.ds(sid * _SU, _SU)])

    @pl.when(sid == _NS - 1)
    def _wtail():
        pltpu.sync_copy(m_sh.at[pl.ds(_SU * _NS, _TAIL)],
                        out_hbm.at[cid, pl.ds(_SU * _NS, _TAIL)])


def kernel(features, distances, cutoffs, rbfs, idx_i, idx_j, params):
    del distances
    blocks = params["blocks"]
    gs = _edge_g_call(rbfs, cutoffs, [p["W_desc"] for p in blocks])
    x = features
    outs = []
    for b in range(NB):
        p = blocks[b]
        xi, y = _atom_pre_call(x, p)
        m01 = _edge_pass_kernel()(gs[b], y, idx_i, idx_j)
        x = _atom_post_call(m01, xi, x, p)
        outs.append(x)
    return jnp.stack(outs, axis=0)


# speedup vs baseline: 1.0748x; 1.0748x over previous
import functools
import math

import jax
import jax.numpy as jnp
import numpy as np
from jax import lax
from jax.experimental import pallas as pl
from jax.experimental.pallas import tpu as pltpu
from jax.experimental.pallas import tpu_sc as plsc

N_ATOMS = 10000
N_PAIRS = 320000
F = 128
R = 64
NB = 5
NRI = 3
NRF = 2
_LOG2 = math.log(2.0)

_NC = 2
_NS = 16
_NW = _NC * _NS
_C = 80
_EW = N_PAIRS // _NW
_NCH = _EW // _C
_NQ = (_NCH - 1) // 4
_SU = 624
_TAIL = N_ATOMS - _SU * _NS

_PERM = np.empty((F,), dtype=np.int32)
for _l in range(F // 32):
    for _k in range(16):
        _PERM[32 * _l + 2 * _k] = 32 * _l + _k
        _PERM[32 * _l + 2 * _k + 1] = 32 * _l + 16 + _k


def _ssp(v):
    return jnp.maximum(v, 0.0) + jnp.log1p(jnp.exp(-jnp.abs(v))) - _LOG2


_EG_CHUNK = 6400


def _edge_g_body(d_ref, *refs):
    w_refs = refs[:NB]
    g_refs = refs[NB:]
    d = d_ref[...]
    for b in range(NB):
        g_refs[b][...] = jnp.dot(d, w_refs[b][...],
                                 preferred_element_type=jnp.float32)


def _edge_g_call(desc, w_list):
    grid = (N_PAIRS // _EG_CHUNK,)
    return pl.pallas_call(
        _edge_g_body,
        grid=grid,
        in_specs=[pl.BlockSpec((_EG_CHUNK, R), lambda i: (i, 0))] +
                 [pl.BlockSpec((R, F), lambda i: (0, 0))] * NB,
        out_specs=[pl.BlockSpec((_EG_CHUNK, F), lambda i: (i, 0))] * NB,
        out_shape=[jax.ShapeDtypeStruct((N_PAIRS, F), jnp.float32)] * NB,
    )(desc, *w_list)


def _atom_pre_body(x_ref, wi_ref, bi_ref, wj_ref, bj_ref, xi_ref, y_ref):
    xa = _ssp(x_ref[...])
    xi_ref[...] = _ssp(jnp.dot(xa, wi_ref[...],
                               preferred_element_type=jnp.float32) + bi_ref[...])
    y_ref[...] = _ssp(jnp.dot(xa, wj_ref[...],
                              preferred_element_type=jnp.float32) + bj_ref[...])


def _atom_pre_call(x, p):
    return pl.pallas_call(
        _atom_pre_body,
        out_shape=(jax.ShapeDtypeStruct((N_ATOMS, F), jnp.float32),
                   jax.ShapeDtypeStruct((N_ATOMS, F), jnp.float32)),
    )(x, p["dense_i"]["W"], p["dense_i"]["b"].reshape(1, F),
      p["dense_j"]["W"], p["dense_j"]["b"].reshape(1, F))


def _atom_step_body(has_next, m01_ref, xi_ref, x_ref, *refs):
    n_out = 3 if has_next else 1
    out_refs = refs[-n_out:]
    w = [r[...] for r in refs[:-n_out]]
    k = 0
    m = m01_ref[0] + m01_ref[1] + xi_ref[...]
    for _ in range(NRI):
        w1, b1, w2, b2 = w[k], w[k + 1], w[k + 2], w[k + 3]
        k += 4
        ym = _ssp(m)
        ym = _ssp(jnp.dot(ym, w1, preferred_element_type=jnp.float32) + b1)
        m = m + jnp.dot(ym, w2, preferred_element_type=jnp.float32) + b2
    wo, bo, u = w[k], w[k + 1], w[k + 2]
    k += 3
    m = _ssp(m)
    x = u * x_ref[...] + jnp.dot(m, wo, preferred_element_type=jnp.float32) + bo
    for _ in range(NRF):
        w1, b1, w2, b2 = w[k], w[k + 1], w[k + 2], w[k + 3]
        k += 4
        yx = _ssp(x)
        yx = _ssp(jnp.dot(yx, w1, preferred_element_type=jnp.float32) + b1)
        x = x + jnp.dot(yx, w2, preferred_element_type=jnp.float32) + b2
    out_refs[0][...] = x
    if has_next:
        wi, bi, wj, bj = w[k], w[k + 1], w[k + 2], w[k + 3]
        xa = _ssp(x)
        out_refs[1][...] = _ssp(jnp.dot(xa, wi,
                                        preferred_element_type=jnp.float32) + bi)
        out_refs[2][...] = _ssp(jnp.dot(xa, wj,
                                        preferred_element_type=jnp.float32) + bj)


def _atom_step_call(m01, xi, x, p, pnext):
    ws = []
    for rp in p["res_int"]:
        ws += [rp["d1"]["W"], rp["d1"]["b"].reshape(1, F),
               rp["d2"]["W"], rp["d2"]["b"].reshape(1, F)]
    ws += [p["dense_out"]["W"], p["dense_out"]["b"].reshape(1, F),
           p["u"].reshape(1, F)]
    for rp in p["res_feat"]:
        ws += [rp["d1"]["W"], rp["d1"]["b"].reshape(1, F),
               rp["d2"]["W"], rp["d2"]["b"].reshape(1, F)]
    n_out = 1
    if pnext is not None:
        ws += [pnext["dense_i"]["W"], pnext["dense_i"]["b"].reshape(1, F),
               pnext["dense_j"]["W"], pnext["dense_j"]["b"].reshape(1, F)]
        n_out = 3
    return pl.pallas_call(
        functools.partial(_atom_step_body, pnext is not None),
        out_shape=tuple(jax.ShapeDtypeStruct((N_ATOMS, F), jnp.float32)
                        for _ in range(n_out)),
    )(m01, xi, x, *ws)


@functools.cache
def _edge_pass_kernel():
    mesh = plsc.VectorSubcoreMesh(core_axis_name="c", subcore_axis_name="s")

    @functools.partial(
        pl.kernel,
        mesh=mesh,
        out_type=jax.ShapeDtypeStruct((_NC, N_ATOMS, F), jnp.float32),
        scratch_types=[
            [pltpu.VMEM((_C,), jnp.int32)] * 4,
            [pltpu.VMEM((_C,), jnp.int32)] * 4,
            [pltpu.VMEM((_C, F), jnp.float32)] * 2,
            [pltpu.VMEM((_C, F), jnp.float32)] * 2,
            pltpu.VMEM_SHARED((N_ATOMS, F), jnp.float32),
            [pltpu.SemaphoreType.DMA] * 4,
            [pltpu.SemaphoreType.DMA] * 2,
            [pltpu.SemaphoreType.DMA] * 2,
            [pltpu.SemaphoreType.DMA] * 2,
        ],
    )
    def _edge_pass(g_hbm, y_hbm, ii_hbm, ij_hbm, out_hbm,
                   ii_c, ij_c, g_v, rows_v, m_sh,
                   sem_i, sem_g, sem_r, sem_s):
        return _edge_pass_body(g_hbm, y_hbm, ii_hbm, ij_hbm, out_hbm,
                               ii_c, ij_c, g_v, rows_v, m_sh,
                               sem_i, sem_g, sem_r, sem_s)

    return _edge_pass


def _mul_row(rows, g, i):
    for l in range(F // 16):
        s0 = pl.ds(l * 16, 16)
        rows[i, s0] = rows[i, s0] * g[i, s0]


def _edge_pass_body(g_hbm, y_hbm, ii_hbm, ij_hbm, out_hbm,
                    ii_c, ij_c, g_v, rows_v, m_sh,
                    sem_i, sem_g, sem_r, sem_s):
    cid = lax.axis_index("c")
    sid = lax.axis_index("s")
    wid = cid * _NS + sid
    ebase = wid * _EW

    def _issue_idx(u, s):
        pltpu.async_copy(ii_hbm.at[pl.ds(ebase + u * _C, _C)], ii_c[s],
                         sem_i[s])
        pltpu.async_copy(ij_hbm.at[pl.ds(ebase + u * _C, _C)], ij_c[s],
                         sem_i[s])

    def _wait_idx(s):
        pltpu.make_async_copy(ii_hbm.at[pl.ds(0, _C)], ii_c[s],
                              sem_i[s]).wait()
        pltpu.make_async_copy(ij_hbm.at[pl.ds(0, _C)], ij_c[s],
                              sem_i[s]).wait()

    def _issue_data(u, b, s):
        pltpu.async_copy(g_hbm.at[pl.ds(ebase + u * _C, _C)], g_v[b],
                         sem_g[b])
        pltpu.async_copy(y_hbm.at[ij_c[s]], rows_v[b], sem_r[b])

    def _wait_data(b, s):
        pltpu.make_async_copy(g_hbm.at[pl.ds(0, _C)], g_v[b],
                              sem_g[b]).wait()
        pltpu.make_async_copy(y_hbm.at[ij_c[s]], rows_v[b],
                              sem_r[b]).wait()

    for s in range(3):
        _issue_idx(s, s)

    zero = jnp.zeros((16,), jnp.float32)

    def _zrow(i, _):
        for b in range(2):
            for l in range(F // 16):
                rows_v[b][i, pl.ds(l * 16, 16)] = zero
        return 0

    lax.fori_loop(0, _C, _zrow, 0)
    for r in range(7):
        pltpu.sync_copy(rows_v[r % 2],
                        m_sh.at[pl.ds(sid * _SU + r * _C, _C)])
    pltpu.sync_copy(rows_v[1].at[pl.ds(0, _SU - 7 * _C)],
                    m_sh.at[pl.ds(sid * _SU + 7 * _C, _SU - 7 * _C)])

    @pl.when(sid == _NS - 1)
    def _ztail():
        pltpu.sync_copy(rows_v[0].at[pl.ds(0, _TAIL)],
                        m_sh.at[pl.ds(_SU * _NS, _TAIL)])

    _wait_idx(0)
    _issue_data(0, 0, 0)

    plsc.subcore_barrier()

    def _step(k, b4):
        u = 4 * k + b4
        b = b4 % 2
        bo = 1 - b
        _wait_data(b, b4)

        def _retire():
            pltpu.make_async_copy(rows_v[bo], m_sh.at[ii_c[(b4 + 3) % 4]],
                                  sem_s[bo]).wait()

        if b4 == 0:
            pl.when(k > 0)(_retire)
        else:
            _retire()

        _wait_idx((b4 + 1) % 4)
        _issue_data(u + 1, bo, (b4 + 1) % 4)

        def _prefetch_idx():
            _issue_idx(u + 3, (b4 + 3) % 4)

        if b4 < 2:
            _prefetch_idx()
        else:
            pl.when(k < _NQ - 1)(_prefetch_idx)

        @plsc.parallel_loop(0, _C, 1, unroll=2)
        def _mul(i):
            _mul_row(rows_v[b], g_v[b], i)

        pltpu.async_copy(rows_v[b], m_sh.at[ii_c[b4]], sem_s[b], add=True)

    def _quad(k, _):
        for b4 in range(4):
            _step(k, b4)
        return 0

    lax.fori_loop(0, _NQ, _quad, 0)

    pltpu.make_async_copy(rows_v[1], m_sh.at[ii_c[3]], sem_s[1]).wait()
    _wait_data(0, 0)

    def _mul_last(i, _):
        _mul_row(rows_v[0], g_v[0], i)
        return 0

    lax.fori_loop(0, _C, _mul_last, 0)
    pltpu.sync_copy(rows_v[0], m_sh.at[ii_c[0]], add=True)

    plsc.subcore_barrier()

    pltpu.sync_copy(m_sh.at[pl.ds(sid * _SU, _SU)],
                    out_hbm.at[cid, pl.ds(sid * _SU, _SU)])

    @pl.when(sid == _NS - 1)
    def _wtail():
        pltpu.sync_copy(m_sh.at[pl.ds(_SU * _NS, _TAIL)],
                        out_hbm.at[cid, pl.ds(_SU * _NS, _TAIL)])


def kernel(features, distances, cutoffs, rbfs, idx_i, idx_j, params):
    del distances
    blocks = params["blocks"]
    desc = cutoffs[:, None] * rbfs
    gs = _edge_g_call(desc, [p["W_desc"] for p in blocks])
    x = features
    xi, y = _atom_pre_call(x, blocks[0])
    outs = []
    for b in range(NB):
        m01 = _edge_pass_kernel()(gs[b], y, idx_i, idx_j)
        if b + 1 < NB:
            x, xi, y = _atom_step_call(m01, xi, x, blocks[b], blocks[b + 1])
        else:
            (x,) = _atom_step_call(m01, xi, x, blocks[b], None)
        outs.append(x)
    return jnp.stack(outs, axis=0)
